# Initial kernel scaffold; baseline (speedup 1.0000x reference)
import jax
import jax.numpy as jnp
from jax.experimental import pallas as pl


def kernel(x, weight, bias):
    raise NotImplementedError("write your pallas kernel here")


# speedup vs baseline: 9.7531x; 9.7531x over previous
import jax
import jax.numpy as jnp
from jax import lax
from jax.experimental import pallas as pl
from jax.experimental.pallas import tpu as pltpu


def _conv_ds_kernel(x_ref, w_ref, b_ref, o_ref, carry_ref):
    j = pl.program_id(1)
    c_out, tl = o_ref.shape

    taps = pltpu.bitcast(x_ref[...], jnp.bfloat16)
    p = jnp.dot(w_ref[...], taps, preferred_element_type=jnp.float32)
    acc = p[:c_out, :]
    p0 = p[c_out:, :]

    @pl.when(j == 0)
    def _():
        carry_ref[...] = jnp.zeros_like(carry_ref)

    prev = carry_ref[...]
    carry_ref[...] = p0[:, tl - 1:tl]
    p0_shift = jnp.concatenate([prev, p0[:, :tl - 1]], axis=1)

    o_ref[...] = acc + p0_shift + b_ref[...]


def kernel(x, weight, bias):
    n, c_in, length = x.shape
    c_out = weight.shape[0]
    l_out = (length - 1) // 2 + 1

    tl = None
    for cand in (2048, 1024, 512, 256, 128):
        if l_out % cand == 0:
            tl = cand
            break
    if tl is None:
        tl = min(l_out, 2048)
    l_out_p = -(-l_out // tl) * tl
    if 2 * l_out_p != length:
        x = jnp.pad(x, ((0, 0), (0, 0), (0, 2 * l_out_p - length)))

    x_pack = lax.bitcast_convert_type(
        x.astype(jnp.bfloat16).reshape(n * c_in, l_out_p, 2), jnp.int32)

    w0, w1, w2 = weight[:, :, 0], weight[:, :, 1], weight[:, :, 2]
    w_cr = jnp.stack([w1, w2], axis=-1).reshape(c_out, 2 * c_in)
    w_0z = jnp.stack([jnp.zeros_like(w0), w0], axis=-1).reshape(c_out, 2 * c_in)
    w_all = jnp.concatenate([w_cr, w_0z], axis=0).astype(jnp.bfloat16)
    b_mat = bias.reshape(c_out, 1).astype(jnp.float32)

    gl = l_out_p // tl
    cost = pl.CostEstimate(
        flops=2 * n * l_out_p * (2 * c_in) * (2 * c_out),
        transcendentals=0,
        bytes_accessed=(x_pack.size * 4 + w_all.size * 2
                        + n * c_out * l_out_p * 4),
    )

    out = pl.pallas_call(
        _conv_ds_kernel,
        out_shape=jax.ShapeDtypeStruct((n * c_out, l_out_p), jnp.float32),
        grid=(n, gl),
        in_specs=[
            pl.BlockSpec((c_in, tl), lambda i, j: (i, j)),
            pl.BlockSpec((2 * c_out, 2 * c_in), lambda i, j: (0, 0)),
            pl.BlockSpec((c_out, 1), lambda i, j: (0, 0)),
        ],
        out_specs=pl.BlockSpec((c_out, tl), lambda i, j: (i, j)),
        scratch_shapes=[pltpu.VMEM((c_out, 1), jnp.float32)],
        compiler_params=pltpu.CompilerParams(
            dimension_semantics=("parallel", "arbitrary"),
            vmem_limit_bytes=64 * 1024 * 1024,
        ),
        cost_estimate=cost,
    )(x_pack, w_all, b_mat)

    out = out.reshape(n, c_out, l_out_p)
    if l_out_p != l_out:
        out = out[:, :, :l_out]
    return out
